# baseline (speedup 1.0000x reference)
import functools

import jax
import jax.numpy as jnp
from jax import lax
from jax.experimental import pallas as pl
from jax.experimental.pallas import tpu as pltpu
from jax.experimental.pallas import tpu_sc as plsc

N = 10000
F = 128
E = 320000
NHID = 128
GDIM = 32

NC = 2
NS = 16
NT = NC * NS
CHUNK = 128
CH0 = 160
CH1 = 0
PH = 40
NPH = -(-CH0 // PH)
TOTAL_CH = NS * (CH0 + CH1)
EPAD = (TOTAL_CH + NPH * PH) * CHUNK
ROWS_PER_TILE = 632
NPAD = NS * ROWS_PER_TILE
DW = 128
BN = 1000
GRID = N // BN

_f32 = jnp.float32
_i32 = jnp.int32


def _seg_body(h_hbm, srcp, dstp, aggp, src_v, dst_v,
              rows0, rows1, agg_sh, g0, g1):
  cid = lax.axis_index("c")
  sid = lax.axis_index("s")

  is0 = cid == 0
  nch = jnp.where(is0, CH0, CH1)
  base_c = jnp.where(is0, sid * CH0, NS * CH0 + sid * CH1)

  zeros16 = jnp.zeros((16,), _f32)

  def _zrow(i, carry):
    for l in range(8):
      rows0[i, pl.ds(l * 16, 16)] = zeros16
    return carry

  lax.fori_loop(0, CHUNK, _zrow, 0)

  base = sid * ROWS_PER_TILE
  for k in range(4):
    pltpu.sync_copy(rows0, agg_sh.at[pl.ds(base + k * CHUNK, CHUNK)])
  pltpu.sync_copy(rows0.at[pl.ds(0, ROWS_PER_TILE - 4 * CHUNK)],
                  agg_sh.at[pl.ds(base + 4 * CHUNK, ROWS_PER_TILE - 4 * CHUNK)])

  plsc.subcore_barrier()

  for ph in range(NPH):
    off = ph * PH
    n_ph = jnp.clip(nch - off, 0, PH)
    pltpu.sync_copy(srcp.at[pl.ds(base_c + off, PH)], src_v)
    pltpu.sync_copy(dstp.at[pl.ds(base_c + off, PH)], dst_v)

    def _pair(k, carry):
      j0 = 2 * k
      j1 = 2 * k + 1
      cp0 = pltpu.async_copy(h_hbm.at[src_v.at[j0]], rows0, g0)
      cp1 = pltpu.async_copy(h_hbm.at[src_v.at[j1]], rows1, g1)
      cp0.wait()
      pltpu.sync_copy(rows0, agg_sh.at[dst_v.at[j0]], add=True)
      cp1.wait()
      pltpu.sync_copy(rows1, agg_sh.at[dst_v.at[j1]], add=True)
      return carry

    lax.fori_loop(0, n_ph // 2, _pair, 0)

  plsc.subcore_barrier()

  pltpu.sync_copy(agg_sh.at[pl.ds(base, ROWS_PER_TILE)],
                  aggp.at[cid, pl.ds(base, ROWS_PER_TILE)])


_seg_call = pl.kernel(
    _seg_body,
    out_type=(jax.ShapeDtypeStruct((NC, NPAD, NHID), _f32),),
    mesh=plsc.VectorSubcoreMesh(core_axis_name="c", subcore_axis_name="s"),
    scratch_types=(
        pltpu.VMEM((PH, CHUNK), _i32),
        pltpu.VMEM((PH, CHUNK), _i32),
        pltpu.VMEM((CHUNK, NHID), _f32),
        pltpu.VMEM((CHUNK, NHID), _f32),
        pltpu.VMEM_SHARED((NPAD, NHID), _f32),
        pltpu.SemaphoreType.DMA,
        pltpu.SemaphoreType.DMA,
    ),
)


DCH = TOTAL_CH // NT


def _deg_body(dstp, degp, dst_v, ones_v, z_v, deg_sh, sem):
  cid = lax.axis_index("c")
  sid = lax.axis_index("s")
  wid = cid * NS + sid

  pltpu.sync_copy(dstp.at[pl.ds(wid * DCH, DCH)], dst_v)

  zeros16 = jnp.zeros((16,), _f32)
  ones16 = jnp.full((16,), 1.0, _f32)

  def _zrow(i, carry):
    for l in range(8):
      ones_v[i, pl.ds(l * 16, 16)] = ones16
      z_v[i, pl.ds(l * 16, 16)] = zeros16
    return carry

  lax.fori_loop(0, CHUNK, _zrow, 0)

  base = sid * ROWS_PER_TILE
  for k in range(4):
    pltpu.sync_copy(z_v, deg_sh.at[pl.ds(base + k * CHUNK, CHUNK)])
  pltpu.sync_copy(z_v.at[pl.ds(0, ROWS_PER_TILE - 4 * CHUNK)],
                  deg_sh.at[pl.ds(base + 4 * CHUNK, ROWS_PER_TILE - 4 * CHUNK)])

  plsc.subcore_barrier()

  def _chunk(j, carry):
    pltpu.sync_copy(ones_v, deg_sh.at[dst_v.at[j]], add=True)
    return carry

  lax.fori_loop(0, DCH, _chunk, 0)

  plsc.subcore_barrier()

  pltpu.sync_copy(deg_sh.at[pl.ds(base, ROWS_PER_TILE)],
                  degp.at[cid, pl.ds(base, ROWS_PER_TILE)])


_deg_call = pl.kernel(
    _deg_body,
    out_type=(jax.ShapeDtypeStruct((NC, NPAD, DW), _f32),),
    mesh=plsc.VectorSubcoreMesh(core_axis_name="c", subcore_axis_name="s"),
    scratch_types=(
        pltpu.VMEM((DCH, CHUNK), _i32),
        pltpu.VMEM((CHUNK, DW), _f32),
        pltpu.VMEM((CHUNK, DW), _f32),
        pltpu.VMEM_SHARED((NPAD, DW), _f32),
        pltpu.SemaphoreType.DMA,
    ),
)


def _layer_body(h_ref, aggp_ref, degp_ref, ws_ref, wn_ref, b_ref,
                out_ref, rmax_ref, rsum_ref):
  i = pl.program_id(0)
  h = h_ref[...]
  agg = aggp_ref[0] + aggp_ref[1]
  deg = degp_ref[0, :, 0:1] + degp_ref[1, :, 0:1]
  scale = 1.0 / jnp.maximum(deg, 1.0)
  a = agg * scale
  hp = jnp.dot(h, ws_ref[...], preferred_element_type=_f32)
  hp += jnp.dot(a, wn_ref[...], preferred_element_type=_f32)
  hp = jnp.maximum(hp + b_ref[...], 0.0)
  out_ref[...] = hp
  bmax = jnp.broadcast_to(jnp.max(hp, axis=0), (8, NHID))
  bsum = jnp.broadcast_to(jnp.sum(hp, axis=0), (8, NHID))

  @pl.when(i == 0)
  def _():
    rmax_ref[...] = bmax
    rsum_ref[...] = bsum

  @pl.when(i > 0)
  def _():
    rmax_ref[...] = jnp.maximum(rmax_ref[...], bmax)
    rsum_ref[...] = rsum_ref[...] + bsum


_layer_call = pl.pallas_call(
    _layer_body,
    grid=(GRID,),
    in_specs=[
        pl.BlockSpec((BN, NHID), lambda i: (i, 0)),
        pl.BlockSpec((NC, BN, NHID), lambda i: (0, i, 0)),
        pl.BlockSpec((NC, BN, DW), lambda i: (0, i, 0)),
        pl.BlockSpec((NHID, NHID), lambda i: (0, 0)),
        pl.BlockSpec((NHID, NHID), lambda i: (0, 0)),
        pl.BlockSpec((1, NHID), lambda i: (0, 0)),
    ],
    out_specs=[
        pl.BlockSpec((BN, NHID), lambda i: (i, 0)),
        pl.BlockSpec((8, NHID), lambda i: (0, 0)),
        pl.BlockSpec((8, NHID), lambda i: (0, 0)),
    ],
    out_shape=[
        jax.ShapeDtypeStruct((N, NHID), _f32),
        jax.ShapeDtypeStruct((8, NHID), _f32),
        jax.ShapeDtypeStruct((8, NHID), _f32),
    ],
)


def _head_body(m1, s1, m2, s2, m3, s3, wl1, bl1, wl2, bl2, out_ref):
  rmax = m1[0:1] + m2[0:1] + m3[0:1]
  rmean = (s1[0:1] + s2[0:1] + s3[0:1]) * (1.0 / N)
  r = jnp.concatenate([rmax, rmean], axis=1)
  z = jnp.dot(r, wl1[...], preferred_element_type=_f32) + bl1[...]
  z = jnp.maximum(z, 0.0)
  z = jnp.dot(z, wl2[...], preferred_element_type=_f32) + bl2[...]
  out_ref[...] = jnp.maximum(z, 0.0)


_head_call = pl.pallas_call(
    _head_body,
    out_shape=jax.ShapeDtypeStruct((1, GDIM), _f32),
)


@jax.jit
def kernel(x, edge_index, W1s, W1n, b1, W2s, W2n, b2, W3s, W3n, b3,
           Wl1, bl1, Wl2, bl2):
  src = edge_index[0]
  dst = edge_index[1]
  pad = EPAD - E
  src_p = jnp.concatenate([src, jnp.zeros((pad,), _i32)])
  src_p = src_p.reshape(EPAD // CHUNK, CHUNK)
  dst_p = jnp.concatenate([dst, jnp.full((pad,), N, _i32)])
  dst_p = dst_p.reshape(EPAD // CHUNK, CHUNK)

  b1r = b1.reshape(1, NHID)
  b2r = b2.reshape(1, NHID)
  b3r = b3.reshape(1, NHID)

  (degp,) = _deg_call(dst_p)
  (agg1p,) = _seg_call(x, src_p, dst_p)
  h1, m1, s1 = _layer_call(x, agg1p, degp, W1s, W1n, b1r)
  (agg2p,) = _seg_call(h1, src_p, dst_p)
  h2, m2, s2 = _layer_call(h1, agg2p, degp, W2s, W2n, b2r)
  (agg3p,) = _seg_call(h2, src_p, dst_p)
  _, m3, s3 = _layer_call(h2, agg3p, degp, W3s, W3n, b3r)
  return _head_call(m1, s1, m2, s2, m3, s3, Wl1, bl1.reshape(1, NHID),
                    Wl2, bl2.reshape(1, GDIM))


# speedup vs baseline: 1.5408x; 1.5408x over previous
import functools

import jax
import jax.numpy as jnp
from jax import lax
from jax.experimental import pallas as pl
from jax.experimental.pallas import tpu as pltpu
from jax.experimental.pallas import tpu_sc as plsc

N = 10000
F = 128
E = 320000
NHID = 128
GDIM = 32

NC = 2
NS = 16
NT = NC * NS
CHUNK = 64
CH0 = 304
CH1 = 16
PH = 40
NPH = -(-CH0 // PH)
TOTAL_CH = NS * (CH0 + CH1)
EPAD = (TOTAL_CH + NPH * PH) * CHUNK
ROWS_PER_TILE = 632
NPAD = NS * ROWS_PER_TILE
ZF = ROWS_PER_TILE // CHUNK
ZR = ROWS_PER_TILE % CHUNK
DW = 128
BN = 1000
GRID = N // BN

_f32 = jnp.float32
_i32 = jnp.int32


def _seg_body(h_hbm, srcp, dstp, aggp, src_v, dst_v,
              rows0, rows1, rows2, rows3, agg_sh, g0, g1, g2, g3):
  rows = (rows0, rows1, rows2, rows3)
  gsem = (g0, g1, g2, g3)
  cid = lax.axis_index("c")
  sid = lax.axis_index("s")

  is0 = cid == 0
  nch = jnp.where(is0, CH0, CH1)
  base_c = jnp.where(is0, sid * CH0, NS * CH0 + sid * CH1)

  zeros16 = jnp.zeros((16,), _f32)

  def _zrow(i, carry):
    for l in range(8):
      rows0[i, pl.ds(l * 16, 16)] = zeros16
    return carry

  lax.fori_loop(0, CHUNK, _zrow, 0)

  base = sid * ROWS_PER_TILE
  for k in range(ZF):
    pltpu.sync_copy(rows0, agg_sh.at[pl.ds(base + k * CHUNK, CHUNK)])
  pltpu.sync_copy(rows0.at[pl.ds(0, ZR)],
                  agg_sh.at[pl.ds(base + ZF * CHUNK, ZR)])

  plsc.subcore_barrier()

  for ph in range(NPH):
    off = ph * PH
    n_ph = jnp.clip(nch - off, 0, PH)
    pltpu.sync_copy(srcp.at[pl.ds(base_c + off, PH)], src_v)
    pltpu.sync_copy(dstp.at[pl.ds(base_c + off, PH)], dst_v)

    def _quad(k, carry):
      cps = [pltpu.async_copy(h_hbm.at[src_v.at[4 * k + b]], rows[b], gsem[b])
             for b in range(4)]
      for b in range(4):
        cps[b].wait()
        pltpu.sync_copy(rows[b], agg_sh.at[dst_v.at[4 * k + b]], add=True)
      return carry

    lax.fori_loop(0, n_ph // 4, _quad, 0)

  plsc.subcore_barrier()

  pltpu.sync_copy(agg_sh.at[pl.ds(base, ROWS_PER_TILE)],
                  aggp.at[cid, pl.ds(base, ROWS_PER_TILE)])


_seg_call = pl.kernel(
    _seg_body,
    out_type=(jax.ShapeDtypeStruct((NC, NPAD, NHID), _f32),),
    mesh=plsc.VectorSubcoreMesh(core_axis_name="c", subcore_axis_name="s"),
    scratch_types=(
        pltpu.VMEM((PH, CHUNK), _i32),
        pltpu.VMEM((PH, CHUNK), _i32),
        pltpu.VMEM((CHUNK, NHID), _f32),
        pltpu.VMEM((CHUNK, NHID), _f32),
        pltpu.VMEM((CHUNK, NHID), _f32),
        pltpu.VMEM((CHUNK, NHID), _f32),
        pltpu.VMEM_SHARED((NPAD, NHID), _f32),
        pltpu.SemaphoreType.DMA,
        pltpu.SemaphoreType.DMA,
        pltpu.SemaphoreType.DMA,
        pltpu.SemaphoreType.DMA,
    ),
)


DCH = TOTAL_CH // NT


def _deg_body(dstp, degp, dst_v, ones_v, z_v, deg_sh, sem):
  cid = lax.axis_index("c")
  sid = lax.axis_index("s")
  wid = cid * NS + sid

  pltpu.sync_copy(dstp.at[pl.ds(wid * DCH, DCH)], dst_v)

  zeros16 = jnp.zeros((16,), _f32)
  ones16 = jnp.full((16,), 1.0, _f32)

  def _zrow(i, carry):
    for l in range(8):
      ones_v[i, pl.ds(l * 16, 16)] = ones16
      z_v[i, pl.ds(l * 16, 16)] = zeros16
    return carry

  lax.fori_loop(0, CHUNK, _zrow, 0)

  base = sid * ROWS_PER_TILE
  for k in range(ZF):
    pltpu.sync_copy(z_v, deg_sh.at[pl.ds(base + k * CHUNK, CHUNK)])
  pltpu.sync_copy(z_v.at[pl.ds(0, ZR)],
                  deg_sh.at[pl.ds(base + ZF * CHUNK, ZR)])

  plsc.subcore_barrier()

  def _chunk(j, carry):
    pltpu.sync_copy(ones_v, deg_sh.at[dst_v.at[j]], add=True)
    return carry

  lax.fori_loop(0, DCH, _chunk, 0)

  plsc.subcore_barrier()

  pltpu.sync_copy(deg_sh.at[pl.ds(base, ROWS_PER_TILE)],
                  degp.at[cid, pl.ds(base, ROWS_PER_TILE)])


_deg_call = pl.kernel(
    _deg_body,
    out_type=(jax.ShapeDtypeStruct((NC, NPAD, DW), _f32),),
    mesh=plsc.VectorSubcoreMesh(core_axis_name="c", subcore_axis_name="s"),
    scratch_types=(
        pltpu.VMEM((DCH, CHUNK), _i32),
        pltpu.VMEM((CHUNK, DW), _f32),
        pltpu.VMEM((CHUNK, DW), _f32),
        pltpu.VMEM_SHARED((NPAD, DW), _f32),
        pltpu.SemaphoreType.DMA,
    ),
)


def _layer_body(h_ref, aggp_ref, degp_ref, ws_ref, wn_ref, b_ref,
                out_ref, rmax_ref, rsum_ref):
  i = pl.program_id(0)
  h = h_ref[...]
  agg = aggp_ref[0] + aggp_ref[1]
  deg = degp_ref[0, :, 0:1] + degp_ref[1, :, 0:1]
  scale = 1.0 / jnp.maximum(deg, 1.0)
  a = agg * scale
  hp = jnp.dot(h, ws_ref[...], preferred_element_type=_f32)
  hp += jnp.dot(a, wn_ref[...], preferred_element_type=_f32)
  hp = jnp.maximum(hp + b_ref[...], 0.0)
  out_ref[...] = hp
  bmax = jnp.broadcast_to(jnp.max(hp, axis=0), (8, NHID))
  bsum = jnp.broadcast_to(jnp.sum(hp, axis=0), (8, NHID))

  @pl.when(i == 0)
  def _():
    rmax_ref[...] = bmax
    rsum_ref[...] = bsum

  @pl.when(i > 0)
  def _():
    rmax_ref[...] = jnp.maximum(rmax_ref[...], bmax)
    rsum_ref[...] = rsum_ref[...] + bsum


_layer_call = pl.pallas_call(
    _layer_body,
    grid=(GRID,),
    in_specs=[
        pl.BlockSpec((BN, NHID), lambda i: (i, 0)),
        pl.BlockSpec((NC, BN, NHID), lambda i: (0, i, 0)),
        pl.BlockSpec((NC, BN, DW), lambda i: (0, i, 0)),
        pl.BlockSpec((NHID, NHID), lambda i: (0, 0)),
        pl.BlockSpec((NHID, NHID), lambda i: (0, 0)),
        pl.BlockSpec((1, NHID), lambda i: (0, 0)),
    ],
    out_specs=[
        pl.BlockSpec((BN, NHID), lambda i: (i, 0)),
        pl.BlockSpec((8, NHID), lambda i: (0, 0)),
        pl.BlockSpec((8, NHID), lambda i: (0, 0)),
    ],
    out_shape=[
        jax.ShapeDtypeStruct((N, NHID), _f32),
        jax.ShapeDtypeStruct((8, NHID), _f32),
        jax.ShapeDtypeStruct((8, NHID), _f32),
    ],
)


def _head_body(m1, s1, m2, s2, m3, s3, wl1, bl1, wl2, bl2, out_ref):
  rmax = m1[0:1] + m2[0:1] + m3[0:1]
  rmean = (s1[0:1] + s2[0:1] + s3[0:1]) * (1.0 / N)
  r = jnp.concatenate([rmax, rmean], axis=1)
  z = jnp.dot(r, wl1[...], preferred_element_type=_f32) + bl1[...]
  z = jnp.maximum(z, 0.0)
  z = jnp.dot(z, wl2[...], preferred_element_type=_f32) + bl2[...]
  out_ref[...] = jnp.maximum(z, 0.0)


_head_call = pl.pallas_call(
    _head_body,
    out_shape=jax.ShapeDtypeStruct((1, GDIM), _f32),
)


@jax.jit
def kernel(x, edge_index, W1s, W1n, b1, W2s, W2n, b2, W3s, W3n, b3,
           Wl1, bl1, Wl2, bl2):
  src = edge_index[0]
  dst = edge_index[1]
  pad = EPAD - E
  src_p = jnp.concatenate([src, jnp.zeros((pad,), _i32)])
  src_p = src_p.reshape(EPAD // CHUNK, CHUNK)
  dst_p = jnp.concatenate([dst, jnp.full((pad,), N, _i32)])
  dst_p = dst_p.reshape(EPAD // CHUNK, CHUNK)

  b1r = b1.reshape(1, NHID)
  b2r = b2.reshape(1, NHID)
  b3r = b3.reshape(1, NHID)

  (degp,) = _deg_call(dst_p)
  (agg1p,) = _seg_call(x, src_p, dst_p)
  h1, m1, s1 = _layer_call(x, agg1p, degp, W1s, W1n, b1r)
  (agg2p,) = _seg_call(h1, src_p, dst_p)
  h2, m2, s2 = _layer_call(h1, agg2p, degp, W2s, W2n, b2r)
  (agg3p,) = _seg_call(h2, src_p, dst_p)
  _, m3, s3 = _layer_call(h2, agg3p, degp, W3s, W3n, b3r)
  return _head_call(m1, s1, m2, s2, m3, s3, Wl1, bl1.reshape(1, NHID),
                    Wl2, bl2.reshape(1, GDIM))


# speedup vs baseline: 1.5523x; 1.0075x over previous
import functools

import jax
import jax.numpy as jnp
from jax import lax
from jax.experimental import pallas as pl
from jax.experimental.pallas import tpu as pltpu
from jax.experimental.pallas import tpu_sc as plsc

N = 10000
F = 128
E = 320000
NHID = 128
GDIM = 32

NC = 2
NS = 16
NT = NC * NS
CHUNK = 128
CH0 = 152
CH1 = 8
PH = 40
NPH = -(-CH0 // PH)
TOTAL_CH = NS * (CH0 + CH1)
EPAD = (TOTAL_CH + NPH * PH) * CHUNK
ROWS_PER_TILE = 632
NPAD = NS * ROWS_PER_TILE
DW = 128
BN = 1000
GRID = N // BN

_f32 = jnp.float32
_i32 = jnp.int32


def _seg_body(h_hbm, srcp, dstp, aggp, src_v, dst_v,
              rows0, rows1, agg_sh, g0, g1):
  cid = lax.axis_index("c")
  sid = lax.axis_index("s")

  is0 = cid == 0
  nch = jnp.where(is0, CH0, CH1)
  base_c = jnp.where(is0, sid * CH0, NS * CH0 + sid * CH1)

  zeros16 = jnp.zeros((16,), _f32)

  def _zrow(i, carry):
    for l in range(8):
      rows0[i, pl.ds(l * 16, 16)] = zeros16
    return carry

  lax.fori_loop(0, CHUNK, _zrow, 0)

  base = sid * ROWS_PER_TILE
  for k in range(4):
    pltpu.sync_copy(rows0, agg_sh.at[pl.ds(base + k * CHUNK, CHUNK)])
  pltpu.sync_copy(rows0.at[pl.ds(0, ROWS_PER_TILE - 4 * CHUNK)],
                  agg_sh.at[pl.ds(base + 4 * CHUNK, ROWS_PER_TILE - 4 * CHUNK)])

  plsc.subcore_barrier()

  for ph in range(NPH):
    off = ph * PH
    n_ph = jnp.clip(nch - off, 0, PH)
    pltpu.sync_copy(srcp.at[pl.ds(base_c + off, PH)], src_v)
    pltpu.sync_copy(dstp.at[pl.ds(base_c + off, PH)], dst_v)

    def _pair(k, carry):
      j0 = 2 * k
      j1 = 2 * k + 1
      cp0 = pltpu.async_copy(h_hbm.at[src_v.at[j0]], rows0, g0)
      cp1 = pltpu.async_copy(h_hbm.at[src_v.at[j1]], rows1, g1)
      cp0.wait()
      pltpu.sync_copy(rows0, agg_sh.at[dst_v.at[j0]], add=True)
      cp1.wait()
      pltpu.sync_copy(rows1, agg_sh.at[dst_v.at[j1]], add=True)
      return carry

    lax.fori_loop(0, n_ph // 2, _pair, 0)

  plsc.subcore_barrier()

  pltpu.sync_copy(agg_sh.at[pl.ds(base, ROWS_PER_TILE)],
                  aggp.at[cid, pl.ds(base, ROWS_PER_TILE)])


_seg_call = pl.kernel(
    _seg_body,
    out_type=(jax.ShapeDtypeStruct((NC, NPAD, NHID), _f32),),
    mesh=plsc.VectorSubcoreMesh(core_axis_name="c", subcore_axis_name="s"),
    scratch_types=(
        pltpu.VMEM((PH, CHUNK), _i32),
        pltpu.VMEM((PH, CHUNK), _i32),
        pltpu.VMEM((CHUNK, NHID), _f32),
        pltpu.VMEM((CHUNK, NHID), _f32),
        pltpu.VMEM_SHARED((NPAD, NHID), _f32),
        pltpu.SemaphoreType.DMA,
        pltpu.SemaphoreType.DMA,
    ),
)


DCH = TOTAL_CH // NT


def _deg_body(dstp, degp, dst_v, ones_v, z_v, deg_sh, sem):
  cid = lax.axis_index("c")
  sid = lax.axis_index("s")
  wid = cid * NS + sid

  pltpu.sync_copy(dstp.at[pl.ds(wid * DCH, DCH)], dst_v)

  zeros16 = jnp.zeros((16,), _f32)
  ones16 = jnp.full((16,), 1.0, _f32)

  def _zrow(i, carry):
    for l in range(8):
      ones_v[i, pl.ds(l * 16, 16)] = ones16
      z_v[i, pl.ds(l * 16, 16)] = zeros16
    return carry

  lax.fori_loop(0, CHUNK, _zrow, 0)

  base = sid * ROWS_PER_TILE
  for k in range(4):
    pltpu.sync_copy(z_v, deg_sh.at[pl.ds(base + k * CHUNK, CHUNK)])
  pltpu.sync_copy(z_v.at[pl.ds(0, ROWS_PER_TILE - 4 * CHUNK)],
                  deg_sh.at[pl.ds(base + 4 * CHUNK, ROWS_PER_TILE - 4 * CHUNK)])

  plsc.subcore_barrier()

  def _chunk(j, carry):
    pltpu.sync_copy(ones_v, deg_sh.at[dst_v.at[j]], add=True)
    return carry

  lax.fori_loop(0, DCH, _chunk, 0)

  plsc.subcore_barrier()

  pltpu.sync_copy(deg_sh.at[pl.ds(base, ROWS_PER_TILE)],
                  degp.at[cid, pl.ds(base, ROWS_PER_TILE)])


_deg_call = pl.kernel(
    _deg_body,
    out_type=(jax.ShapeDtypeStruct((NC, NPAD, DW), _f32),),
    mesh=plsc.VectorSubcoreMesh(core_axis_name="c", subcore_axis_name="s"),
    scratch_types=(
        pltpu.VMEM((DCH, CHUNK), _i32),
        pltpu.VMEM((CHUNK, DW), _f32),
        pltpu.VMEM((CHUNK, DW), _f32),
        pltpu.VMEM_SHARED((NPAD, DW), _f32),
        pltpu.SemaphoreType.DMA,
    ),
)


def _layer_body(h_ref, aggp_ref, degp_ref, ws_ref, wn_ref, b_ref,
                out_ref, rmax_ref, rsum_ref):
  i = pl.program_id(0)
  h = h_ref[...]
  agg = aggp_ref[0] + aggp_ref[1]
  deg = degp_ref[0, :, 0:1] + degp_ref[1, :, 0:1]
  scale = 1.0 / jnp.maximum(deg, 1.0)
  a = agg * scale
  hp = jnp.dot(h, ws_ref[...], preferred_element_type=_f32)
  hp += jnp.dot(a, wn_ref[...], preferred_element_type=_f32)
  hp = jnp.maximum(hp + b_ref[...], 0.0)
  out_ref[...] = hp
  bmax = jnp.broadcast_to(jnp.max(hp, axis=0), (8, NHID))
  bsum = jnp.broadcast_to(jnp.sum(hp, axis=0), (8, NHID))

  @pl.when(i == 0)
  def _():
    rmax_ref[...] = bmax
    rsum_ref[...] = bsum

  @pl.when(i > 0)
  def _():
    rmax_ref[...] = jnp.maximum(rmax_ref[...], bmax)
    rsum_ref[...] = rsum_ref[...] + bsum


_layer_call = pl.pallas_call(
    _layer_body,
    grid=(GRID,),
    in_specs=[
        pl.BlockSpec((BN, NHID), lambda i: (i, 0)),
        pl.BlockSpec((NC, BN, NHID), lambda i: (0, i, 0)),
        pl.BlockSpec((NC, BN, DW), lambda i: (0, i, 0)),
        pl.BlockSpec((NHID, NHID), lambda i: (0, 0)),
        pl.BlockSpec((NHID, NHID), lambda i: (0, 0)),
        pl.BlockSpec((1, NHID), lambda i: (0, 0)),
    ],
    out_specs=[
        pl.BlockSpec((BN, NHID), lambda i: (i, 0)),
        pl.BlockSpec((8, NHID), lambda i: (0, 0)),
        pl.BlockSpec((8, NHID), lambda i: (0, 0)),
    ],
    out_shape=[
        jax.ShapeDtypeStruct((N, NHID), _f32),
        jax.ShapeDtypeStruct((8, NHID), _f32),
        jax.ShapeDtypeStruct((8, NHID), _f32),
    ],
)


def _head_body(m1, s1, m2, s2, m3, s3, wl1, bl1, wl2, bl2, out_ref):
  rmax = m1[0:1] + m2[0:1] + m3[0:1]
  rmean = (s1[0:1] + s2[0:1] + s3[0:1]) * (1.0 / N)
  r = jnp.concatenate([rmax, rmean], axis=1)
  z = jnp.dot(r, wl1[...], preferred_element_type=_f32) + bl1[...]
  z = jnp.maximum(z, 0.0)
  z = jnp.dot(z, wl2[...], preferred_element_type=_f32) + bl2[...]
  out_ref[...] = jnp.maximum(z, 0.0)


_head_call = pl.pallas_call(
    _head_body,
    out_shape=jax.ShapeDtypeStruct((1, GDIM), _f32),
)


@jax.jit
def kernel(x, edge_index, W1s, W1n, b1, W2s, W2n, b2, W3s, W3n, b3,
           Wl1, bl1, Wl2, bl2):
  src = edge_index[0]
  dst = edge_index[1]
  pad = EPAD - E
  src_p = jnp.concatenate([src, jnp.zeros((pad,), _i32)])
  src_p = src_p.reshape(EPAD // CHUNK, CHUNK)
  dst_p = jnp.concatenate([dst, jnp.full((pad,), N, _i32)])
  dst_p = dst_p.reshape(EPAD // CHUNK, CHUNK)

  b1r = b1.reshape(1, NHID)
  b2r = b2.reshape(1, NHID)
  b3r = b3.reshape(1, NHID)

  (degp,) = _deg_call(dst_p)
  (agg1p,) = _seg_call(x, src_p, dst_p)
  h1, m1, s1 = _layer_call(x, agg1p, degp, W1s, W1n, b1r)
  (agg2p,) = _seg_call(h1, src_p, dst_p)
  h2, m2, s2 = _layer_call(h1, agg2p, degp, W2s, W2n, b2r)
  (agg3p,) = _seg_call(h2, src_p, dst_p)
  _, m3, s3 = _layer_call(h2, agg3p, degp, W3s, W3n, b3r)
  return _head_call(m1, s1, m2, s2, m3, s3, Wl1, bl1.reshape(1, NHID),
                    Wl2, bl2.reshape(1, GDIM))
